# baseline (speedup 1.0000x reference)
import functools

import jax
import jax.numpy as jnp
from jax import lax
from jax.experimental import pallas as pl
from jax.experimental.pallas import tpu as pltpu
from jax.experimental.pallas import tpu_sc as plsc

_BN = 12800
_LANES = 16


@functools.lru_cache(maxsize=None)
def _make_sc_gather(n_pad: int, n_tbl: int):
    info = plsc.get_sparse_core_info()
    nc = 1
    nw = nc * info.num_subcores
    chunk = n_pad // nw
    assert n_pad % nw == 0 and chunk % 8 == 0 and chunk % _LANES == 0

    mesh = plsc.VectorSubcoreMesh(
        core_axis_name="c", subcore_axis_name="s", num_cores=nc)

    @functools.partial(
        pl.kernel,
        out_type=jax.ShapeDtypeStruct((n_pad,), jnp.int32),
        mesh=mesh,
        compiler_params=pltpu.CompilerParams(needs_layout_passes=False),
        scratch_types=[
            pltpu.VMEM((n_tbl,), jnp.int32),
            pltpu.VMEM((chunk,), jnp.int32),
            pltpu.VMEM((chunk,), jnp.int32),
        ],
    )
    def sc_gather(idx_hbm, tbl_hbm, out_hbm, tbl_v, idx_v, val_v):
        wid = lax.axis_index("s") * nc + lax.axis_index("c")
        base = wid * chunk
        pltpu.sync_copy(tbl_hbm, tbl_v)
        pltpu.sync_copy(idx_hbm.at[pl.ds(base, chunk)], idx_v)

        def body(i, carry):
            sl = pl.ds(i * _LANES, _LANES)
            val_v[sl] = plsc.load_gather(tbl_v, [idx_v[sl]])
            return carry

        lax.fori_loop(0, chunk // _LANES, body, 0, unroll=4)
        pltpu.sync_copy(val_v, out_hbm.at[pl.ds(base, chunk)])

    return sc_gather


def _tc_body(ads_ref, x_ref, w2_ref, b_ref, out_ref):
    xb = x_ref[...]
    w2 = w2_ref[...]
    yt = lax.dot_general(
        w2, xb, (((1,), (1,)), ((), ())),
        preferred_element_type=jnp.float32,
    )
    ads = ads_ref[0]
    eid = lax.broadcasted_iota(jnp.int32, yt.shape, 0)
    out_ref[...] = jnp.where(eid == ads, yt + b_ref[...], 0.0)


def kernel(x, batch, dataset_ids, W, b):
    n, d = x.shape
    e, _, o = W.shape
    batch = batch.astype(jnp.int32)
    dataset_ids = dataset_ids.astype(jnp.int32)

    nb = pl.cdiv(n, _BN)
    n_pad = nb * _BN
    batch_p = jnp.pad(batch, (0, n_pad - n))
    ads = _make_sc_gather(n_pad, dataset_ids.shape[0])(batch_p, dataset_ids)
    ads3 = ads.reshape(nb, 1, _BN)

    w2 = W[:, :, 0]
    out = pl.pallas_call(
        _tc_body,
        grid=(nb,),
        in_specs=[
            pl.BlockSpec((1, 1, _BN), lambda i: (i, 0, 0)),
            pl.BlockSpec((_BN, d), lambda i: (i, 0)),
            pl.BlockSpec((e, d), lambda i: (0, 0)),
            pl.BlockSpec((e, o), lambda i: (0, 0)),
        ],
        out_specs=pl.BlockSpec((e, _BN), lambda i: (0, i)),
        out_shape=jax.ShapeDtypeStruct((e, n), jnp.float32),
    )(ads3, x, w2, b)
    return out[:, :, None]


# speedup vs baseline: 1.0151x; 1.0151x over previous
import functools

import jax
import jax.numpy as jnp
from jax import lax
from jax.experimental import pallas as pl
from jax.experimental.pallas import tpu as pltpu
from jax.experimental.pallas import tpu_sc as plsc

_BN = 12800
_LANES = 16


@functools.lru_cache(maxsize=None)
def _make_sc_gather(n: int, n_pad: int, n_tbl: int):
    info = plsc.get_sparse_core_info()
    nc = 1
    nw = nc * info.num_subcores
    chunk = n_pad // nw
    tail = n - (nw - 1) * chunk
    assert n_pad % nw == 0 and chunk % 8 == 0 and chunk % _LANES == 0
    assert 0 < tail <= chunk and tail % 8 == 0 and tail % _LANES == 0

    mesh = plsc.VectorSubcoreMesh(
        core_axis_name="c", subcore_axis_name="s", num_cores=nc)

    @functools.partial(
        pl.kernel,
        out_type=jax.ShapeDtypeStruct((n_pad,), jnp.int32),
        mesh=mesh,
        compiler_params=pltpu.CompilerParams(needs_layout_passes=False),
        scratch_types=[
            pltpu.VMEM((n_tbl,), jnp.int32),
            pltpu.VMEM((chunk,), jnp.int32),
            pltpu.VMEM((chunk,), jnp.int32),
        ],
    )
    def sc_gather(idx_hbm, tbl_hbm, out_hbm, tbl_v, idx_v, val_v):
        wid = lax.axis_index("s") * nc + lax.axis_index("c")
        base = wid * chunk
        is_last = wid == nw - 1
        pltpu.sync_copy(tbl_hbm, tbl_v)

        def body(i, carry):
            sl = pl.ds(i * _LANES, _LANES)
            val_v[sl] = plsc.load_gather(tbl_v, [idx_v[sl]])
            return carry

        @pl.when(jnp.logical_not(is_last))
        def _():
            pltpu.sync_copy(idx_hbm.at[pl.ds(base, chunk)], idx_v)
            lax.fori_loop(0, chunk // _LANES, body, 0)
            pltpu.sync_copy(val_v, out_hbm.at[pl.ds(base, chunk)])

        @pl.when(is_last)
        def _():
            pltpu.sync_copy(idx_hbm.at[pl.ds(base, tail)],
                            idx_v.at[pl.ds(0, tail)])
            lax.fori_loop(0, tail // _LANES, body, 0)
            pltpu.sync_copy(val_v.at[pl.ds(0, tail)],
                            out_hbm.at[pl.ds(base, tail)])

    return sc_gather


def _tc_body(ads_ref, x_ref, w2_ref, b_ref, out_ref):
    xb = x_ref[...]
    w2 = w2_ref[...]
    yt = lax.dot_general(
        w2, xb, (((1,), (1,)), ((), ())),
        preferred_element_type=jnp.float32,
    )
    ads = ads_ref[0]
    eid = lax.broadcasted_iota(jnp.int32, yt.shape, 0)
    out_ref[...] = jnp.where(eid == ads, yt + b_ref[...], 0.0)


def kernel(x, batch, dataset_ids, W, b):
    n, d = x.shape
    e, _, o = W.shape
    batch = batch.astype(jnp.int32)
    dataset_ids = dataset_ids.astype(jnp.int32)

    nb = pl.cdiv(n, _BN)
    n_pad = nb * _BN
    ads = _make_sc_gather(n, n_pad, dataset_ids.shape[0])(batch, dataset_ids)
    ads3 = ads.reshape(nb, 1, _BN)

    w2 = W[:, :, 0]
    out = pl.pallas_call(
        _tc_body,
        grid=(nb,),
        in_specs=[
            pl.BlockSpec((1, 1, _BN), lambda i: (i, 0, 0)),
            pl.BlockSpec((_BN, d), lambda i: (i, 0)),
            pl.BlockSpec((e, d), lambda i: (0, 0)),
            pl.BlockSpec((e, o), lambda i: (0, 0)),
        ],
        out_specs=pl.BlockSpec((e, _BN), lambda i: (0, i)),
        out_shape=jax.ShapeDtypeStruct((e, n), jnp.float32),
    )(ads3, x, w2, b)
    return out[:, :, None]


# speedup vs baseline: 1.0189x; 1.0038x over previous
import functools

import jax
import jax.numpy as jnp
from jax import lax
from jax.experimental import pallas as pl
from jax.experimental.pallas import tpu as pltpu
from jax.experimental.pallas import tpu_sc as plsc

_BN = 12800
_LANES = 16


@functools.lru_cache(maxsize=None)
def _make_sc_gather(n_pad: int, n_tbl: int):
    info = plsc.get_sparse_core_info()
    nc = 1
    nw = nc * info.num_subcores
    chunk = n_pad // nw
    assert n_pad % nw == 0 and chunk % 8 == 0 and chunk % _LANES == 0

    mesh = plsc.VectorSubcoreMesh(
        core_axis_name="c", subcore_axis_name="s", num_cores=nc)

    @functools.partial(
        pl.kernel,
        out_type=jax.ShapeDtypeStruct((n_pad,), jnp.int32),
        mesh=mesh,
        compiler_params=pltpu.CompilerParams(needs_layout_passes=False),
        scratch_types=[
            pltpu.VMEM((n_tbl,), jnp.int32),
            pltpu.VMEM((chunk,), jnp.int32),
            pltpu.VMEM((chunk,), jnp.int32),
        ],
    )
    def sc_gather(idx_hbm, tbl_hbm, out_hbm, tbl_v, idx_v, val_v):
        wid = lax.axis_index("s") * nc + lax.axis_index("c")
        base = wid * chunk
        pltpu.sync_copy(tbl_hbm, tbl_v)
        pltpu.sync_copy(idx_hbm.at[pl.ds(base, chunk)], idx_v)

        def body(i, carry):
            sl = pl.ds(i * _LANES, _LANES)
            val_v[sl] = plsc.load_gather(tbl_v, [idx_v[sl]])
            return carry

        lax.fori_loop(0, chunk // _LANES, body, 0)
        pltpu.sync_copy(val_v, out_hbm.at[pl.ds(base, chunk)])

    return sc_gather


def _tc_body(ads_ref, x_ref, w2_ref, b_ref, out_ref):
    xb = x_ref[...]
    w2 = w2_ref[...]
    yt = lax.dot_general(
        w2, xb, (((1,), (1,)), ((), ())),
        preferred_element_type=jnp.float32,
    )
    ads = ads_ref[0]
    eid = lax.broadcasted_iota(jnp.int32, yt.shape, 0)
    out_ref[...] = jnp.where(eid == ads, yt + b_ref[...], 0.0)


def kernel(x, batch, dataset_ids, W, b):
    n, d = x.shape
    e, _, o = W.shape
    batch = batch.astype(jnp.int32)
    dataset_ids = dataset_ids.astype(jnp.int32)

    nb = pl.cdiv(n, _BN)
    n_pad = nb * _BN
    batch_p = jnp.pad(batch, (0, n_pad - n))
    ads = _make_sc_gather(n_pad, dataset_ids.shape[0])(batch_p, dataset_ids)
    ads3 = ads.reshape(nb, 1, _BN)

    w2 = W[:, :, 0]
    out = pl.pallas_call(
        _tc_body,
        grid=(nb,),
        in_specs=[
            pl.BlockSpec((1, 1, _BN), lambda i: (i, 0, 0)),
            pl.BlockSpec((_BN, d), lambda i: (i, 0)),
            pl.BlockSpec((e, d), lambda i: (0, 0)),
            pl.BlockSpec((e, o), lambda i: (0, 0)),
        ],
        out_specs=pl.BlockSpec((e, _BN), lambda i: (0, i)),
        out_shape=jax.ShapeDtypeStruct((e, n), jnp.float32),
    )(ads3, x, w2, b)
    return out[:, :, None]
